# baseline (speedup 1.0000x reference)
import functools

import jax
import jax.numpy as jnp
from jax import lax
from jax.experimental import pallas as pl
from jax.experimental.pallas import tpu as pltpu
from jax.experimental.pallas import tpu_sc as plsc

_NUM_FIELDS = 26
_VOCAB = 100000
_EMBED = 32
_BATCH = 16384

_INFO = plsc.get_sparse_core_info()
_NC = _INFO.num_cores
_NS = _INFO.num_subcores
_NW = _NC * _NS


def _make_embed(nf):
    @functools.partial(
        pl.kernel,
        mesh=plsc.VectorSubcoreMesh(core_axis_name="c", subcore_axis_name="s"),
        out_type=tuple(
            jax.ShapeDtypeStruct((_EMBED, _BATCH), jnp.float32)
            for _ in range(nf)
        ),
        scratch_types=[
            pltpu.VMEM((2, _BATCH), jnp.int32),
            pltpu.VMEM((2, _BATCH), jnp.float32),
            pltpu.SemaphoreType.DMA,
            pltpu.SemaphoreType.DMA,
            pltpu.SemaphoreType.DMA,
        ],
        compiler_params=pltpu.CompilerParams(
            use_tc_tiling_on_sc=False, needs_layout_passes=False
        ),
    )
    def _embed(table_t_hbm, x_t_hbm, *refs):
        outs = refs[:nf]
        idx_v, val_v, isem, gsem, osem = refs[nf:]
        e = lax.axis_index("s") * _NC + lax.axis_index("c")

        pltpu.async_copy(x_t_hbm.at[0], idx_v.at[0], isem).wait()
        if nf > 1:
            pltpu.async_copy(x_t_hbm.at[1], idx_v.at[1], isem)
        pltpu.async_copy(table_t_hbm.at[e].at[idx_v.at[0]], val_v.at[0], gsem)
        for f in range(nf):
            b = f % 2
            nb = (f + 1) % 2
            pltpu.make_async_copy(table_t_hbm.at[0].at[idx_v.at[b]],
                                  val_v.at[b], gsem).wait()
            if f + 1 < nf:
                pltpu.make_async_copy(x_t_hbm.at[0], idx_v.at[0], isem).wait()
                if f >= 1:
                    pltpu.make_async_copy(
                        val_v.at[0], outs[0].at[e], osem).wait()
                pltpu.async_copy(
                    table_t_hbm.at[(f + 1) * _EMBED + e].at[idx_v.at[nb]],
                    val_v.at[nb], gsem)
                if f + 2 < nf:
                    pltpu.async_copy(x_t_hbm.at[f + 2], idx_v.at[b], isem)
            pltpu.async_copy(val_v.at[b], outs[f].at[e], osem)
        pltpu.make_async_copy(val_v.at[0], outs[0].at[e], osem).wait()
        if nf > 1:
            pltpu.make_async_copy(val_v.at[0], outs[0].at[e], osem).wait()

    return _embed


_HALF = _NUM_FIELDS // 2
_embed_lo = _make_embed(_HALF)
_embed_hi = _make_embed(_NUM_FIELDS - _HALF)


def kernel(tables, x):
    table_t = tables.transpose(0, 2, 1)
    x_t = x.T
    lo = _embed_lo(
        table_t[:_HALF].reshape(_HALF * _EMBED, _VOCAB), x_t[:_HALF])
    hi = _embed_hi(
        table_t[_HALF:].reshape((_NUM_FIELDS - _HALF) * _EMBED, _VOCAB),
        x_t[_HALF:])
    return tuple(o.T for o in (lo + hi))


# speedup vs baseline: 1.0380x; 1.0380x over previous
import functools

import jax
import jax.numpy as jnp
from jax import lax
from jax.experimental import pallas as pl
from jax.experimental.pallas import tpu as pltpu
from jax.experimental.pallas import tpu_sc as plsc

_NUM_FIELDS = 26
_VOCAB = 100000
_EMBED = 32
_BATCH = 16384

_INFO = plsc.get_sparse_core_info()
_NC = _INFO.num_cores
_NS = _INFO.num_subcores
_NW = _NC * _NS


@functools.partial(
    pl.kernel,
    mesh=plsc.VectorSubcoreMesh(core_axis_name="c", subcore_axis_name="s"),
    out_type=tuple(
        jax.ShapeDtypeStruct((_EMBED, _BATCH), jnp.float32)
        for _ in range(_NUM_FIELDS)
    ),
    scratch_types=[
        pltpu.VMEM((2, _BATCH), jnp.int32),
        pltpu.VMEM((2, _BATCH), jnp.float32),
        pltpu.SemaphoreType.DMA,
        pltpu.SemaphoreType.DMA,
        pltpu.SemaphoreType.DMA,
    ],
    compiler_params=pltpu.CompilerParams(
        use_tc_tiling_on_sc=False, needs_layout_passes=False
    ),
)
def _embed_all(table_t_hbm, x_t_hbm, *refs):
    outs = refs[:_NUM_FIELDS]
    idx_v, val_v, isem, gsem, osem = refs[_NUM_FIELDS:]
    e = lax.axis_index("s") * _NC + lax.axis_index("c")

    pltpu.async_copy(x_t_hbm.at[0], idx_v.at[0], isem).wait()
    pltpu.async_copy(x_t_hbm.at[1], idx_v.at[1], isem)
    pltpu.async_copy(table_t_hbm.at[e].at[idx_v.at[0]], val_v.at[0], gsem)
    for f in range(_NUM_FIELDS):
        b = f % 2
        nb = (f + 1) % 2
        pltpu.make_async_copy(table_t_hbm.at[0].at[idx_v.at[b]],
                              val_v.at[b], gsem).wait()
        if f + 1 < _NUM_FIELDS:
            pltpu.make_async_copy(x_t_hbm.at[0], idx_v.at[0], isem).wait()
            if f >= 1:
                pltpu.make_async_copy(val_v.at[0], outs[0].at[e], osem).wait()
            pltpu.async_copy(
                table_t_hbm.at[(f + 1) * _EMBED + e].at[idx_v.at[nb]],
                val_v.at[nb], gsem)
            if f + 2 < _NUM_FIELDS:
                pltpu.async_copy(x_t_hbm.at[f + 2], idx_v.at[b], isem)
        pltpu.async_copy(val_v.at[b], outs[f].at[e], osem)
    pltpu.make_async_copy(val_v.at[0], outs[0].at[e], osem).wait()
    pltpu.make_async_copy(val_v.at[0], outs[0].at[e], osem).wait()


def kernel(tables, x):
    table_t = tables.transpose(0, 2, 1).reshape(_NUM_FIELDS * _EMBED, _VOCAB)
    x_t = x.T
    outs_t = _embed_all(table_t, x_t)
    return tuple(o.T for o in outs_t)
